# baseline (speedup 1.0000x reference)
import functools

import jax
import jax.numpy as jnp
from jax import lax
from jax.experimental import pallas as pl
from jax.experimental.pallas import tpu as pltpu
from jax.experimental.pallas import tpu_sc as plsc

D_MODEL = 1024
NUM_CORES = 2
NUM_SUBCORES = 16
NUM_WORKERS = NUM_CORES * NUM_SUBCORES
CHUNK = 32
NBUF = 3


@functools.partial(jax.jit, static_argnames=("b_per_w", "n_chunks"))
def _gather_rows(pe, idx, *, b_per_w, n_chunks):
    total = idx.shape[0]
    mesh = plsc.VectorSubcoreMesh(core_axis_name="c", subcore_axis_name="s")

    @functools.partial(
        pl.kernel,
        out_type=jax.ShapeDtypeStruct((total, D_MODEL), jnp.float32),
        mesh=mesh,
        scratch_types=[
            pltpu.VMEM((b_per_w,), jnp.int32),
            pltpu.VMEM((NBUF, CHUNK, D_MODEL), jnp.float32),
            [pltpu.SemaphoreType.DMA] * NBUF,
            [pltpu.SemaphoreType.DMA] * NBUF,
        ],
    )
    def body(pe_hbm, idx_hbm, out_hbm, idx_v, rows_v, sems_g, sems_s):
        wid = lax.axis_index("s") * NUM_CORES + lax.axis_index("c")
        base = wid * b_per_w
        pltpu.sync_copy(idx_hbm.at[pl.ds(base, b_per_w)], idx_v)

        def gather_start(chunk, b):
            pltpu.async_copy(
                pe_hbm.at[idx_v.at[pl.ds(chunk * CHUNK, CHUNK)]],
                rows_v.at[b],
                sems_g[b],
            )

        def gather_wait(b):
            pltpu.make_async_copy(
                pe_hbm.at[idx_v.at[pl.ds(0, CHUNK)]], rows_v.at[b], sems_g[b]
            ).wait()

        def scatter_start(chunk, b):
            pltpu.async_copy(
                rows_v.at[b],
                out_hbm.at[pl.ds(base + chunk * CHUNK, CHUNK)],
                sems_s[b],
            )

        def scatter_wait(b):
            pltpu.make_async_copy(
                rows_v.at[b], out_hbm.at[pl.ds(base, CHUNK)], sems_s[b]
            ).wait()

        lookahead = NBUF - 1
        for j in range(min(lookahead, n_chunks)):
            gather_start(j, j % NBUF)
        for j in range(n_chunks):
            nxt = j + lookahead
            if nxt < n_chunks:
                b_nxt = nxt % NBUF
                if nxt >= NBUF:
                    scatter_wait(b_nxt)
                gather_start(nxt, b_nxt)
            b = j % NBUF
            gather_wait(b)
            scatter_start(j, b)
        for j in range(max(0, n_chunks - NBUF), n_chunks):
            scatter_wait(j % NBUF)

    return body(pe, idx)


def kernel(position_ids, pe):
    idx = position_ids.reshape(-1)
    total = idx.shape[0]
    b_per_w = total // NUM_WORKERS
    out = _gather_rows(pe, idx, b_per_w=b_per_w, n_chunks=b_per_w // CHUNK)
    return out.reshape(position_ids.shape + (pe.shape[1],))
